# baseline (speedup 1.0000x reference)
import functools

import jax
import jax.numpy as jnp
from jax import lax
from jax.experimental import pallas as pl
from jax.experimental.pallas import tpu as pltpu
from jax.experimental.pallas import tpu_sc as plsc

NC = 2
NS = 16
NW = NC * NS
CHUNK = 1600


def _emb_body(b_per_w, nchunk, ids_hbm, table_hbm, out3_hbm,
              idx0, idx1, rows0, rows1,
              isem0, isem1, gsem0, gsem1, osem0, osem1):
    wid = lax.axis_index("s") * NC + lax.axis_index("c")
    batch, hist, embed = out3_hbm.shape
    rows_per_chunk = CHUNK // hist
    wrows = b_per_w // hist

    wbase = wid * b_per_w

    def idx_copy(g, buf, sem):
        base = pl.multiple_of(wbase + g * CHUNK, CHUNK)
        return pltpu.make_async_copy(ids_hbm.at[pl.ds(base, CHUNK)], buf, sem)

    def out_copy(g, buf, sem):
        bq = pl.multiple_of(wid * wrows + g * rows_per_chunk, rows_per_chunk)
        return pltpu.make_async_copy(
            buf, out3_hbm.at[pl.ds(bq, rows_per_chunk)], sem)

    class _Gat:
        def __init__(self, idxbuf, rowbuf, sem):
            self.descs = [
                pltpu.make_async_copy(
                    table_hbm.at[idxbuf.at[pl.ds(r * hist, hist)]],
                    rowbuf.at[r], sem)
                for r in range(rows_per_chunk)
            ]

        def start(self):
            for d in self.descs:
                d.start()

        def wait(self):
            for d in self.descs:
                d.wait()

    def gat_copy(idxbuf, rowbuf, sem):
        return _Gat(idxbuf, rowbuf, sem)

    idx_copy(0, idx0, isem0).start()
    idx_copy(1, idx1, isem1).start()
    idx_copy(0, idx0, isem0).wait()
    gat_copy(idx0, rows0, gsem0).start()
    idx_copy(1, idx1, isem1).wait()
    gat_copy(idx1, rows1, gsem1).start()
    gat_copy(idx0, rows0, gsem0).wait()
    out_copy(0, rows0, osem0).start()
    idx_copy(2, idx0, isem0).start()
    gat_copy(idx1, rows1, gsem1).wait()
    out_copy(1, rows1, osem1).start()
    idx_copy(3, idx1, isem1).start()

    def body(g2, carry):
        g = 2 * g2
        idx_copy(g, idx0, isem0).wait()
        out_copy(g - 2, rows0, osem0).wait()
        gat_copy(idx0, rows0, gsem0).start()
        idx_copy(g + 1, idx1, isem1).wait()
        out_copy(g - 1, rows1, osem1).wait()
        gat_copy(idx1, rows1, gsem1).start()
        gat_copy(idx0, rows0, gsem0).wait()
        out_copy(g, rows0, osem0).start()
        idx_copy(g + 2, idx0, isem0).start()
        gat_copy(idx1, rows1, gsem1).wait()
        out_copy(g + 1, rows1, osem1).start()
        idx_copy(g + 3, idx1, isem1).start()
        return carry

    lax.fori_loop(1, nchunk // 2 - 1, body, 0, unroll=False)

    g = nchunk - 2
    idx_copy(g, idx0, isem0).wait()
    out_copy(g - 2, rows0, osem0).wait()
    gat_copy(idx0, rows0, gsem0).start()
    idx_copy(g + 1, idx1, isem1).wait()
    out_copy(g - 1, rows1, osem1).wait()
    gat_copy(idx1, rows1, gsem1).start()
    gat_copy(idx0, rows0, gsem0).wait()
    out_copy(g, rows0, osem0).start()
    gat_copy(idx1, rows1, gsem1).wait()
    out_copy(g + 1, rows1, osem1).start()
    out_copy(g, rows0, osem0).wait()
    out_copy(g + 1, rows1, osem1).wait()


def kernel(ids, weight):
    batch, hist = ids.shape
    vocab, embed = weight.shape
    b_total = batch * hist
    assert b_total % (NW * CHUNK) == 0
    b_per_w = b_total // NW
    nchunk = b_per_w // CHUNK
    assert nchunk >= 4 and nchunk % 2 == 0

    ids_flat = ids.reshape(b_total).astype(jnp.int32)

    mesh = plsc.VectorSubcoreMesh(core_axis_name="c", subcore_axis_name="s")
    emb = pl.kernel(
        functools.partial(_emb_body, b_per_w, nchunk),
        out_type=jax.ShapeDtypeStruct((batch, hist, embed), jnp.float32),
        mesh=mesh,
        scratch_types=[
            pltpu.VMEM((CHUNK,), jnp.int32),
            pltpu.VMEM((CHUNK,), jnp.int32),
            pltpu.VMEM((CHUNK // hist, hist, embed), jnp.float32),
            pltpu.VMEM((CHUNK // hist, hist, embed), jnp.float32),
            pltpu.SemaphoreType.DMA,
            pltpu.SemaphoreType.DMA,
            pltpu.SemaphoreType.DMA,
            pltpu.SemaphoreType.DMA,
            pltpu.SemaphoreType.DMA,
            pltpu.SemaphoreType.DMA,
        ],
        compiler_params=pltpu.CompilerParams(use_tc_tiling_on_sc=False),
    )
    return emb(ids_flat, weight)


# speedup vs baseline: 1.0355x; 1.0355x over previous
import functools

import jax
import jax.numpy as jnp
from jax import lax
from jax.experimental import pallas as pl
from jax.experimental.pallas import tpu as pltpu
from jax.experimental.pallas import tpu_sc as plsc

NC = 2
NS = 16
NW = NC * NS
HB = 256
L = 16


def _emb_body(brows_per_w, ids_hbm, table_hbm, out_hbm,
              idsraw, idst, rows0, rows1,
              isem, gsem0, gsem1, osem0, osem1):
    wid = lax.axis_index("s") * NC + lax.axis_index("c")
    hist, batch, embed = out_hbm.shape
    wbase = wid * brows_per_w
    nhb = brows_per_w // HB

    lane = lax.iota(jnp.int32, L)

    def load_ids(hb):
        b0 = pl.multiple_of((wbase + hb * HB) * hist, HB * hist)
        return pltpu.make_async_copy(
            ids_hbm.at[pl.ds(b0, HB * hist)], idsraw, isem)

    def transpose_ids():
        def h_body(h, carry):
            def c_body(c, carry2):
                src = (c * L + lane) * hist + h
                vals = plsc.load_gather(idsraw, [src])
                idst[pl.ds(h * HB + c * L, L)] = vals
                return carry2
            return lax.fori_loop(0, HB // L, c_body, carry, unroll=4)
        lax.fori_loop(0, hist, h_body, 0, unroll=False)

    def gat(h, buf, sem):
        return pltpu.make_async_copy(
            table_hbm.at[idst.at[pl.ds(h * HB, HB)]], buf, sem)

    def out_copy(hb, h, buf, sem):
        b0 = pl.multiple_of(wbase + hb * HB, HB)
        return pltpu.make_async_copy(
            buf, out_hbm.at[h, pl.ds(b0, HB)], sem)

    def half_block(hb):
        load_ids(hb).wait()
        transpose_ids()

        gat(0, rows0, gsem0).start()
        gat(1, rows1, gsem1).start()
        gat(0, rows0, gsem0).wait()
        out_copy(hb, 0, rows0, osem0).start()
        gat(1, rows1, gsem1).wait()
        out_copy(hb, 1, rows1, osem1).start()

        def body(h2, carry):
            h = 2 * h2
            out_copy(hb, h - 2, rows0, osem0).wait()
            gat(h, rows0, gsem0).start()
            out_copy(hb, h - 1, rows1, osem1).wait()
            gat(h + 1, rows1, gsem1).start()
            gat(h, rows0, gsem0).wait()
            out_copy(hb, h, rows0, osem0).start()
            gat(h + 1, rows1, gsem1).wait()
            out_copy(hb, h + 1, rows1, osem1).start()
            return carry

        lax.fori_loop(1, hist // 2, body, 0, unroll=False)
        out_copy(hb, hist - 2, rows0, osem0).wait()
        out_copy(hb, hist - 1, rows1, osem1).wait()

    load_ids(0).start()
    for hb in range(nhb):
        half_block(hb)
        if hb + 1 < nhb:
            load_ids(hb + 1).start()


def kernel(ids, weight):
    batch, hist = ids.shape
    vocab, embed = weight.shape
    assert batch % (NW * HB) == 0 and hist % 2 == 0 and HB % L == 0
    brows_per_w = batch // NW

    ids_flat = ids.reshape(batch * hist).astype(jnp.int32)

    mesh = plsc.VectorSubcoreMesh(core_axis_name="c", subcore_axis_name="s")
    emb = pl.kernel(
        functools.partial(_emb_body, brows_per_w),
        out_type=jax.ShapeDtypeStruct((hist, batch, embed), jnp.float32),
        mesh=mesh,
        scratch_types=[
            pltpu.VMEM((HB * hist,), jnp.int32),
            pltpu.VMEM((hist * HB,), jnp.int32),
            pltpu.VMEM((HB, embed), jnp.float32),
            pltpu.VMEM((HB, embed), jnp.float32),
            pltpu.SemaphoreType.DMA,
            pltpu.SemaphoreType.DMA,
            pltpu.SemaphoreType.DMA,
            pltpu.SemaphoreType.DMA,
            pltpu.SemaphoreType.DMA,
        ],
        compiler_params=pltpu.CompilerParams(
            use_tc_tiling_on_sc=False, needs_layout_passes=False),
    )
    out_l = emb(ids_flat, weight)
    return out_l.transpose(1, 0, 2)
